# baseline (speedup 1.0000x reference)
import functools

import jax
import jax.numpy as jnp
from jax.experimental import pallas as pl
from jax.experimental.pallas import tpu as pltpu


def _emb_kernel(row_ref, col_ref, pose_ref, p_ref, m_ref, *, h, w, c):
    colT = col_ref[1:w + 1, :].T
    rowT = row_ref[1:h + 1, :].T
    m_ref[0, :c, :] = jnp.broadcast_to(colT[:, None, :], (c, h, w)).reshape(c, h * w)
    m_ref[0, c:, :] = jnp.broadcast_to(rowT[:, :, None], (c, h, w)).reshape(c, h * w)
    pv = pose_ref[0, :]
    p_ref[0, 0, :c] = pv
    p_ref[0, 0, c:] = pv


def kernel(x, row_embed, col_embed, pose_token_embed):
    b = x.shape[0]
    h, w = x.shape[-2], x.shape[-1]
    c = row_embed.shape[1]
    dt = row_embed.dtype

    kfn = functools.partial(_emb_kernel, h=h, w=w, c=c)

    p_emb, m_flat = pl.pallas_call(
        kfn,
        grid=(b,),
        in_specs=[
            pl.BlockSpec(row_embed.shape, lambda i: (0, 0)),
            pl.BlockSpec(col_embed.shape, lambda i: (0, 0)),
            pl.BlockSpec(pose_token_embed.shape, lambda i: (0, 0)),
        ],
        out_specs=[
            pl.BlockSpec((1, 1, 2 * c), lambda i: (i, 0, 0)),
            pl.BlockSpec((1, 2 * c, h * w), lambda i: (i, 0, 0)),
        ],
        out_shape=[
            jax.ShapeDtypeStruct((b, 1, 2 * c), dt),
            jax.ShapeDtypeStruct((b, 2 * c, h * w), dt),
        ],
        compiler_params=pltpu.CompilerParams(
            dimension_semantics=("parallel",),
        ),
    )(row_embed, col_embed, pose_token_embed)

    return p_emb.reshape(b, 2 * c), m_flat.reshape(b, 2 * c, h, w)


# speedup vs baseline: 1.5475x; 1.5475x over previous
import functools

import jax
import jax.numpy as jnp
from jax.experimental import pallas as pl
from jax.experimental.pallas import tpu as pltpu


def _emb_kernel(row_ref, col_ref, pose_ref, p_hbm, m_hbm, m_s, p_s, msem, psem,
                *, b, h, w, c):
    colT = col_ref[1:w + 1, :].T
    rowT = row_ref[1:h + 1, :].T
    m_s[:c, :] = jnp.broadcast_to(colT[:, None, :], (c, h, w)).reshape(c, h * w)
    m_s[c:, :] = jnp.broadcast_to(rowT[:, :, None], (c, h, w)).reshape(c, h * w)
    pv = pose_ref[0, :]
    p_s[:, :c] = jnp.broadcast_to(pv[None, :], (b, c))
    p_s[:, c:] = jnp.broadcast_to(pv[None, :], (b, c))

    pcopy = pltpu.make_async_copy(p_s, p_hbm, psem)
    pcopy.start(priority=1)
    mcopies = [pltpu.make_async_copy(m_s, m_hbm.at[i], msem.at[i]) for i in range(b)]
    for i, cp in enumerate(mcopies):
        cp.start(priority=i % 2)
    pcopy.wait()
    for cp in mcopies:
        cp.wait()


def kernel(x, row_embed, col_embed, pose_token_embed):
    b = x.shape[0]
    h, w = x.shape[-2], x.shape[-1]
    c = row_embed.shape[1]
    dt = row_embed.dtype

    kfn = functools.partial(_emb_kernel, b=b, h=h, w=w, c=c)

    p_emb, m_flat = pl.pallas_call(
        kfn,
        in_specs=[
            pl.BlockSpec(memory_space=pltpu.MemorySpace.VMEM),
            pl.BlockSpec(memory_space=pltpu.MemorySpace.VMEM),
            pl.BlockSpec(memory_space=pltpu.MemorySpace.VMEM),
        ],
        out_specs=[
            pl.BlockSpec(memory_space=pltpu.MemorySpace.HBM),
            pl.BlockSpec(memory_space=pltpu.MemorySpace.HBM),
        ],
        out_shape=[
            jax.ShapeDtypeStruct((b, 2 * c), dt),
            jax.ShapeDtypeStruct((b, 2 * c, h * w), dt),
        ],
        scratch_shapes=[
            pltpu.VMEM((2 * c, h * w), dt),
            pltpu.VMEM((b, 2 * c), dt),
            pltpu.SemaphoreType.DMA((b,)),
            pltpu.SemaphoreType.DMA,
        ],
    )(row_embed, col_embed, pose_token_embed)

    return p_emb, m_flat.reshape(b, 2 * c, h, w)
